# baseline (speedup 1.0000x reference)
import functools
import math

import jax
import jax.numpy as jnp
from jax import lax
from jax.experimental import pallas as pl
from jax.experimental.pallas import tpu as pltpu
from jax.experimental.pallas import tpu_sc as plsc

EMB_D = 32
SCALE = math.sqrt(float(EMB_D))

NC, NS, LANES = 2, 16, 16
NW = NC * NS

B_TOTAL = 16384 * 50
BPW = B_TOTAL // NW
CHUNK = 128
NSTEPS = BPW // CHUNK


def _emb_body(idx_hbm, table_hbm, out_hbm, idx_v, rows_v, sem):
    wid = lax.axis_index("s") * NC + lax.axis_index("c")
    base = wid * BPW

    pltpu.sync_copy(idx_hbm.at[wid], idx_v)

    def step(j, _):
        pltpu.async_copy(table_hbm.at[idx_v.at[j]], rows_v, sem).wait()

        def scale_row(i, _):
            rows_v[i, pl.ds(0, LANES)] = rows_v[i, pl.ds(0, LANES)] * SCALE
            rows_v[i, pl.ds(LANES, LANES)] = (
                rows_v[i, pl.ds(LANES, LANES)] * SCALE)
            return 0

        lax.fori_loop(0, CHUNK, scale_row, 0)

        pltpu.sync_copy(rows_v, out_hbm.at[pl.ds(base + j * CHUNK, CHUNK)])
        return 0

    lax.fori_loop(0, NSTEPS, step, 0)


@jax.jit
def _emb(idx, table):
    mesh = plsc.VectorSubcoreMesh(core_axis_name="c", subcore_axis_name="s")
    f = pl.kernel(
        _emb_body,
        out_type=jax.ShapeDtypeStruct((B_TOTAL, EMB_D), jnp.float32),
        mesh=mesh,
        scratch_types=[
            pltpu.VMEM((NSTEPS, CHUNK), jnp.int32),
            pltpu.VMEM((CHUNK, EMB_D), jnp.float32),
            pltpu.SemaphoreType.DMA,
        ],
        compiler_params=pltpu.CompilerParams(use_tc_tiling_on_sc=False),
    )
    return f(idx, table)


def kernel(x, embed_table):
    idx = x.reshape(NW, NSTEPS, CHUNK).astype(jnp.int32)
    out = _emb(idx, embed_table)
    return out.reshape(x.shape[0], x.shape[1], EMB_D)


# speedup vs baseline: 1.1564x; 1.1564x over previous
import math

import jax
import jax.numpy as jnp
from jax import lax
from jax.experimental import pallas as pl
from jax.experimental.pallas import tpu as pltpu
from jax.experimental.pallas import tpu_sc as plsc

EMB_D = 32
SCALE = math.sqrt(float(EMB_D))

NC, NS, LANES = 2, 16, 16
NW = NC * NS

B_TOTAL = 16384 * 50
BPW = B_TOTAL // NW
GIDX = 128
GPC = 4
CROWS = GIDX * GPC
NCHUNK = BPW // CROWS
NSTEPS = BPW // GIDX
ROWS_PER_ITER = 8


def _emb_body(idx_hbm, table_hbm, out_hbm,
              idx_v, in0, in1, out0, out1, gsem0, gsem1, ssem0, ssem1):
    wid = lax.axis_index("s") * NC + lax.axis_index("c")
    base = wid * BPW

    pltpu.sync_copy(idx_hbm.at[wid], idx_v)

    def issue_gathers(c, in_b, gsem_b):
        for g in range(GPC):
            pltpu.async_copy(
                table_hbm.at[idx_v.at[c * GPC + g]],
                in_b.at[pl.ds(g * GIDX, GIDX)],
                gsem_b,
            )

    def drain_gathers(c, in_b, gsem_b):
        for g in range(GPC):
            pltpu.make_async_copy(
                table_hbm.at[idx_v.at[c * GPC + g]],
                in_b.at[pl.ds(g * GIDX, GIDX)],
                gsem_b,
            ).wait()

    def scale_chunk(in_b, out_b):
        def body(it, _):
            for r in range(ROWS_PER_ITER):
                i = it * ROWS_PER_ITER + r
                out_b[i, pl.ds(0, LANES)] = in_b[i, pl.ds(0, LANES)] * SCALE
                out_b[i, pl.ds(LANES, LANES)] = (
                    in_b[i, pl.ds(LANES, LANES)] * SCALE)
            return 0

        lax.fori_loop(0, CROWS // ROWS_PER_ITER, body, 0)

    issue_gathers(0, in0, gsem0)
    issue_gathers(1, in1, gsem1)

    bufs = ((in0, out0, gsem0, ssem0), (in1, out1, gsem1, ssem1))

    def pair(t, _):
        for b in range(2):
            in_b, out_b, gsem_b, ssem_b = bufs[b]
            c = 2 * t + b
            drain_gathers(c, in_b, gsem_b)

            @pl.when(t > 0)
            def _():
                pltpu.make_async_copy(
                    out_b, out_hbm.at[pl.ds(base + c * CROWS, CROWS)], ssem_b
                ).wait()

            scale_chunk(in_b, out_b)

            @pl.when(c + 2 < NCHUNK)
            def _():
                issue_gathers(c + 2, in_b, gsem_b)

            pltpu.async_copy(
                out_b, out_hbm.at[pl.ds(base + c * CROWS, CROWS)], ssem_b)
        return 0

    lax.fori_loop(0, NCHUNK // 2, pair, 0)

    for b in range(2):
        in_b, out_b, gsem_b, ssem_b = bufs[b]
        c = NCHUNK - 2 + b
        pltpu.make_async_copy(
            out_b, out_hbm.at[pl.ds(base + c * CROWS, CROWS)], ssem_b
        ).wait()


@jax.jit
def _emb(idx, table):
    mesh = plsc.VectorSubcoreMesh(core_axis_name="c", subcore_axis_name="s")
    f = pl.kernel(
        _emb_body,
        out_type=jax.ShapeDtypeStruct((B_TOTAL, EMB_D), jnp.float32),
        mesh=mesh,
        scratch_types=[
            pltpu.VMEM((NSTEPS, GIDX), jnp.int32),
            pltpu.VMEM((CROWS, EMB_D), jnp.float32),
            pltpu.VMEM((CROWS, EMB_D), jnp.float32),
            pltpu.VMEM((CROWS, EMB_D), jnp.float32),
            pltpu.VMEM((CROWS, EMB_D), jnp.float32),
            pltpu.SemaphoreType.DMA,
            pltpu.SemaphoreType.DMA,
            pltpu.SemaphoreType.DMA,
            pltpu.SemaphoreType.DMA,
        ],
        compiler_params=pltpu.CompilerParams(use_tc_tiling_on_sc=False),
    )
    return f(idx, table)


def kernel(x, embed_table):
    idx = x.reshape(NW, NSTEPS, GIDX).astype(jnp.int32)
    out = _emb(idx, embed_table)
    return out.reshape(x.shape[0], x.shape[1], EMB_D)


# speedup vs baseline: 1.8906x; 1.6349x over previous
import math

import jax
import jax.numpy as jnp
from jax import lax
from jax.experimental import pallas as pl
from jax.experimental.pallas import tpu as pltpu
from jax.experimental.pallas import tpu_sc as plsc

EMB_D = 32
SCALE = math.sqrt(float(EMB_D))

NC, NS, LANES = 2, 16, 16
NW = NC * NS

N_SEQ, N_TOK = 16384, 50
B_TOTAL = N_SEQ * N_TOK
GIDX = 128
NBLK = B_TOTAL // GIDX
BPW = NBLK // NW
SB = N_SEQ // GIDX


def _emb_body(idx_hbm, table_hbm, out_hbm,
              idx_v, rows0, rows1, t40, t41, gsem0, gsem1, ssem0, ssem1):
    wid = lax.axis_index("s") * NC + lax.axis_index("c")
    g0 = wid * BPW

    pltpu.sync_copy(idx_hbm.at[pl.ds(g0, BPW)], idx_v)

    lane = lax.iota(jnp.int32, LANES)
    dt_lo = lane >> 3
    dp_vec = lane & 7
    dt_hi = dt_lo + 2

    def issue_gather(i, rows_b, gsem_b):
        pltpu.async_copy(table_hbm.at[idx_v.at[i]], rows_b, gsem_b)

    def wait_gather(i, rows_b, gsem_b):
        pltpu.make_async_copy(table_hbm.at[idx_v.at[i]], rows_b, gsem_b).wait()

    def transpose_scale(rows_b, t4_b):
        def body(sp, _):
            spv = jnp.full((LANES,), sp, dtype=jnp.int32)
            lo = rows_b[sp, pl.ds(0, LANES)] * SCALE
            hi = rows_b[sp, pl.ds(LANES, LANES)] * SCALE
            plsc.store_scatter(t4_b, [dt_lo, dp_vec, spv], lo)
            plsc.store_scatter(t4_b, [dt_hi, dp_vec, spv], hi)
            return 0

        lax.fori_loop(0, GIDX, body, 0)

    def issue_stores(j, st, t4_b, ssem_b):
        for dt in range(4):
            pltpu.async_copy(t4_b.at[dt], out_hbm.at[j, dt, st], ssem_b)

    def wait_stores(j, st, t4_b, ssem_b):
        for dt in range(4):
            pltpu.make_async_copy(t4_b.at[dt], out_hbm.at[j, dt, st], ssem_b
                                  ).wait()

    issue_gather(0, rows0, gsem0)
    issue_gather(1, rows1, gsem1)

    bufs = ((rows0, t40, gsem0, ssem0), (rows1, t41, gsem1, ssem1))

    def pair(t, _):
        for b in range(2):
            rows_b, t4_b, gsem_b, ssem_b = bufs[b]
            i = 2 * t + b
            g = g0 + i
            j = g >> 7
            st = g & (SB - 1)
            wait_gather(i, rows_b, gsem_b)

            @pl.when(t > 0)
            def _():
                g_prev = g - 2
                wait_stores(g_prev >> 7, g_prev & (SB - 1), t4_b, ssem_b)

            transpose_scale(rows_b, t4_b)

            @pl.when(i + 2 < BPW)
            def _():
                issue_gather(i + 2, rows_b, gsem_b)

            issue_stores(j, st, t4_b, ssem_b)
        return 0

    lax.fori_loop(0, BPW // 2, pair, 0)

    for b in range(2):
        rows_b, t4_b, gsem_b, ssem_b = bufs[b]
        g = g0 + BPW - 2 + b
        wait_stores(g >> 7, g & (SB - 1), t4_b, ssem_b)


@jax.jit
def _emb(idx2, table):
    mesh = plsc.VectorSubcoreMesh(core_axis_name="c", subcore_axis_name="s")
    f = pl.kernel(
        _emb_body,
        out_type=jax.ShapeDtypeStruct((N_TOK, 4, SB, 8, GIDX), jnp.float32),
        mesh=mesh,
        scratch_types=[
            pltpu.VMEM((BPW, GIDX), jnp.int32),
            pltpu.VMEM((GIDX, EMB_D), jnp.float32),
            pltpu.VMEM((GIDX, EMB_D), jnp.float32),
            pltpu.VMEM((4, 8, GIDX), jnp.float32),
            pltpu.VMEM((4, 8, GIDX), jnp.float32),
            pltpu.SemaphoreType.DMA,
            pltpu.SemaphoreType.DMA,
            pltpu.SemaphoreType.DMA,
            pltpu.SemaphoreType.DMA,
        ],
        compiler_params=pltpu.CompilerParams(
            use_tc_tiling_on_sc=False, needs_layout_passes=False),
    )
    return f(idx2, table)


def kernel(x, embed_table):
    idx2 = x.T.reshape(NBLK, GIDX).astype(jnp.int32)
    out5 = _emb(idx2, embed_table)
    return out5.transpose(2, 4, 0, 1, 3).reshape(N_SEQ, N_TOK, EMB_D)
